# baseline (device time: 19722 ns/iter reference)
import jax
import jax.numpy as jnp
from jax import lax
from jax.experimental import pallas as pl
from jax.experimental.pallas import tpu as pltpu

NC = 8


def kernel(x, dy):
    k, m = x.shape
    _, f = dy.shape
    m_half = m // 2
    f_half = f // 2
    fc = f_half // NC

    def body(x_ref, dy_ref, out_ref, c_send, rs_recv, p2_send, p2_recv,
             dy_half, x_keep, x_send, out_vmem,
             sems1_s, sems1_r, sems2_s, sems2_r, ldma_sems, xdma_sems,
             odma_sems):
        my_x = lax.axis_index("x")
        my_y = lax.axis_index("y")
        is_x0 = my_x == 0

        xs_copy = pltpu.make_async_copy(
            x_ref.at[:, pl.ds((1 - my_y) * m_half, m_half)], x_send,
            xdma_sems.at[0])
        xk_copy = pltpu.make_async_copy(
            x_ref.at[:, pl.ds(my_y * m_half, m_half)], x_keep,
            xdma_sems.at[1])
        xs_copy.start()
        xk_copy.start()
        dy_copies = []
        for c in range(NC):
            cp = pltpu.make_async_copy(
                dy_ref.at[:, pl.ds(my_x * f_half + c * fc, fc)],
                dy_half.at[:, c * fc:(c + 1) * fc], ldma_sems.at[c])
            cp.start()
            dy_copies.append(cp)

        barrier = pltpu.get_barrier_semaphore()
        pl.semaphore_signal(barrier, inc=1, device_id=(1 - my_x, my_y),
                            device_id_type=pl.DeviceIdType.MESH)
        pl.semaphore_signal(barrier, inc=1, device_id=(my_x, 1 - my_y),
                            device_id_type=pl.DeviceIdType.MESH)
        pl.semaphore_wait(barrier, 2)

        xs_copy.wait()
        rdma1 = []
        for c in range(NC):
            dy_copies[c].wait()
            c_send[c] = lax.dot_general(
                x_send[...], dy_half[:, c * fc:(c + 1) * fc],
                (((0,), (0,)), ((), ())),
                preferred_element_type=jnp.float32).astype(jnp.bfloat16)
            r = pltpu.make_async_remote_copy(
                src_ref=c_send.at[c], dst_ref=rs_recv.at[c],
                send_sem=sems1_s.at[c], recv_sem=sems1_r.at[c],
                device_id=(my_x, 1 - my_y),
                device_id_type=pl.DeviceIdType.MESH)
            r.start()
            rdma1.append(r)

        def drain(d):
            rdma2[d].wait_recv()
            lo, hi = d * fc, (d + 1) * fc

            @pl.when(is_x0)
            def _(lo=lo, hi=hi, d=d):
                out_vmem[:, f_half + lo:f_half + hi] = (
                    p2_recv[d].astype(jnp.float32))

            @pl.when(~is_x0)
            def _(lo=lo, hi=hi, d=d):
                out_vmem[:, lo:hi] = p2_recv[d].astype(jnp.float32)

            theirs = pl.ds((1 - my_x) * f_half + lo, fc)
            ocp = pltpu.make_async_copy(
                out_vmem.at[:, theirs], out_ref.at[:, theirs],
                odma_sems.at[NC + d])
            ocp.start()
            out_copies.append(ocp)
            rdma1[d].wait_send()
            rdma2[d].wait_send()

        LAG = 3
        xk_copy.wait()
        rdma2 = []
        out_copies = []
        for c in range(NC):
            keep = lax.dot_general(
                x_keep[...], dy_half[:, c * fc:(c + 1) * fc],
                (((0,), (0,)), ((), ())), preferred_element_type=jnp.float32)
            rdma1[c].wait_recv()
            val = keep + rs_recv[c].astype(jnp.float32)
            p2_send[c] = val.astype(jnp.bfloat16)
            lo, hi = c * fc, (c + 1) * fc

            @pl.when(is_x0)
            def _(lo=lo, hi=hi, val=val):
                out_vmem[:, lo:hi] = val

            @pl.when(~is_x0)
            def _(lo=lo, hi=hi, val=val):
                out_vmem[:, f_half + lo:f_half + hi] = val

            mine = pl.ds(my_x * f_half + lo, fc)
            ocp = pltpu.make_async_copy(
                out_vmem.at[:, mine], out_ref.at[:, mine], odma_sems.at[c])
            ocp.start()
            out_copies.append(ocp)

            r2 = pltpu.make_async_remote_copy(
                src_ref=p2_send.at[c], dst_ref=p2_recv.at[c],
                send_sem=sems2_s.at[c], recv_sem=sems2_r.at[c],
                device_id=(1 - my_x, my_y),
                device_id_type=pl.DeviceIdType.MESH)
            r2.start()
            rdma2.append(r2)
            if c >= LAG:
                drain(c - LAG)

        for d in range(NC - LAG, NC):
            drain(d)

        for ocp in out_copies:
            ocp.wait()

    return pl.pallas_call(
        body,
        out_shape=jax.ShapeDtypeStruct((m_half, f), jnp.float32),
        in_specs=[pl.BlockSpec(memory_space=pl.ANY),
                  pl.BlockSpec(memory_space=pl.ANY)],
        out_specs=pl.BlockSpec(memory_space=pl.ANY),
        scratch_shapes=[
            pltpu.VMEM((NC, m_half, fc), jnp.bfloat16),
            pltpu.VMEM((NC, m_half, fc), jnp.bfloat16),
            pltpu.VMEM((NC, m_half, fc), jnp.bfloat16),
            pltpu.VMEM((NC, m_half, fc), jnp.bfloat16),
            pltpu.VMEM((k, f_half), jnp.float32),
            pltpu.VMEM((k, m_half), jnp.float32),
            pltpu.VMEM((k, m_half), jnp.float32),
            pltpu.VMEM((m_half, f), jnp.float32),
            pltpu.SemaphoreType.DMA((NC,)),
            pltpu.SemaphoreType.DMA((NC,)),
            pltpu.SemaphoreType.DMA((NC,)),
            pltpu.SemaphoreType.DMA((NC,)),
            pltpu.SemaphoreType.DMA((NC,)),
            pltpu.SemaphoreType.DMA((2,)),
            pltpu.SemaphoreType.DMA((2 * NC,)),
        ],
        compiler_params=pltpu.CompilerParams(collective_id=0),
    )(x, dy)


# device time: 19599 ns/iter; 1.0063x vs baseline; 1.0063x over previous
import jax
import jax.numpy as jnp
from jax import lax
from jax.experimental import pallas as pl
from jax.experimental.pallas import tpu as pltpu

NC = 4


def kernel(x, dy):
    k, m = x.shape
    _, f = dy.shape
    m_half = m // 2
    f_half = f // 2
    fc = f_half // NC

    def body(x_ref, dy_ref, out_ref, c_send, rs_recv, p2_send, p2_recv,
             dy_half, x_keep, x_send, out_vmem,
             sems1_s, sems1_r, sems2_s, sems2_r, ldma_sems, xdma_sems,
             odma_sems):
        my_x = lax.axis_index("x")
        my_y = lax.axis_index("y")
        is_x0 = my_x == 0

        xs_copy = pltpu.make_async_copy(
            x_ref.at[:, pl.ds((1 - my_y) * m_half, m_half)], x_send,
            xdma_sems.at[0])
        xk_copy = pltpu.make_async_copy(
            x_ref.at[:, pl.ds(my_y * m_half, m_half)], x_keep,
            xdma_sems.at[1])
        xs_copy.start()
        xk_copy.start()
        dy_copies = []
        for c in range(NC):
            cp = pltpu.make_async_copy(
                dy_ref.at[:, pl.ds(my_x * f_half + c * fc, fc)],
                dy_half.at[:, c * fc:(c + 1) * fc], ldma_sems.at[c])
            cp.start()
            dy_copies.append(cp)

        barrier = pltpu.get_barrier_semaphore()
        pl.semaphore_signal(barrier, inc=1, device_id=(1 - my_x, my_y),
                            device_id_type=pl.DeviceIdType.MESH)
        pl.semaphore_signal(barrier, inc=1, device_id=(my_x, 1 - my_y),
                            device_id_type=pl.DeviceIdType.MESH)
        pl.semaphore_wait(barrier, 2)

        xs_copy.wait()
        rdma1 = []
        for c in range(NC):
            dy_copies[c].wait()
            c_send[c] = lax.dot_general(
                x_send[...], dy_half[:, c * fc:(c + 1) * fc],
                (((0,), (0,)), ((), ())),
                preferred_element_type=jnp.float32).astype(jnp.bfloat16)
            r = pltpu.make_async_remote_copy(
                src_ref=c_send.at[c], dst_ref=rs_recv.at[c],
                send_sem=sems1_s.at[c], recv_sem=sems1_r.at[c],
                device_id=(my_x, 1 - my_y),
                device_id_type=pl.DeviceIdType.MESH)
            r.start()
            rdma1.append(r)

        def drain(d):
            rdma2[d].wait_recv()
            lo, hi = d * fc, (d + 1) * fc

            @pl.when(is_x0)
            def _(lo=lo, hi=hi, d=d):
                out_vmem[:, f_half + lo:f_half + hi] = (
                    p2_recv[d].astype(jnp.float32))

            @pl.when(~is_x0)
            def _(lo=lo, hi=hi, d=d):
                out_vmem[:, lo:hi] = p2_recv[d].astype(jnp.float32)

            theirs = pl.ds((1 - my_x) * f_half + lo, fc)
            ocp = pltpu.make_async_copy(
                out_vmem.at[:, theirs], out_ref.at[:, theirs],
                odma_sems.at[NC + d])
            ocp.start()
            out_copies.append(ocp)
            rdma1[d].wait_send()
            rdma2[d].wait_send()

        LAG = 3
        xk_copy.wait()
        rdma2 = []
        out_copies = []
        for c in range(NC):
            keep = lax.dot_general(
                x_keep[...], dy_half[:, c * fc:(c + 1) * fc],
                (((0,), (0,)), ((), ())), preferred_element_type=jnp.float32)
            rdma1[c].wait_recv()
            val = keep + rs_recv[c].astype(jnp.float32)
            p2_send[c] = val.astype(jnp.bfloat16)
            lo, hi = c * fc, (c + 1) * fc

            @pl.when(is_x0)
            def _(lo=lo, hi=hi, val=val):
                out_vmem[:, lo:hi] = val

            @pl.when(~is_x0)
            def _(lo=lo, hi=hi, val=val):
                out_vmem[:, f_half + lo:f_half + hi] = val

            mine = pl.ds(my_x * f_half + lo, fc)
            ocp = pltpu.make_async_copy(
                out_vmem.at[:, mine], out_ref.at[:, mine], odma_sems.at[c])
            ocp.start()
            out_copies.append(ocp)

            r2 = pltpu.make_async_remote_copy(
                src_ref=p2_send.at[c], dst_ref=p2_recv.at[c],
                send_sem=sems2_s.at[c], recv_sem=sems2_r.at[c],
                device_id=(1 - my_x, my_y),
                device_id_type=pl.DeviceIdType.MESH)
            r2.start()
            rdma2.append(r2)
            if c >= LAG:
                drain(c - LAG)

        for d in range(NC - LAG, NC):
            drain(d)

        for ocp in out_copies:
            ocp.wait()

    return pl.pallas_call(
        body,
        out_shape=jax.ShapeDtypeStruct((m_half, f), jnp.float32),
        in_specs=[pl.BlockSpec(memory_space=pl.ANY),
                  pl.BlockSpec(memory_space=pl.ANY)],
        out_specs=pl.BlockSpec(memory_space=pl.ANY),
        scratch_shapes=[
            pltpu.VMEM((NC, m_half, fc), jnp.bfloat16),
            pltpu.VMEM((NC, m_half, fc), jnp.bfloat16),
            pltpu.VMEM((NC, m_half, fc), jnp.bfloat16),
            pltpu.VMEM((NC, m_half, fc), jnp.bfloat16),
            pltpu.VMEM((k, f_half), jnp.float32),
            pltpu.VMEM((k, m_half), jnp.float32),
            pltpu.VMEM((k, m_half), jnp.float32),
            pltpu.VMEM((m_half, f), jnp.float32),
            pltpu.SemaphoreType.DMA((NC,)),
            pltpu.SemaphoreType.DMA((NC,)),
            pltpu.SemaphoreType.DMA((NC,)),
            pltpu.SemaphoreType.DMA((NC,)),
            pltpu.SemaphoreType.DMA((NC,)),
            pltpu.SemaphoreType.DMA((2,)),
            pltpu.SemaphoreType.DMA((2 * NC,)),
        ],
        compiler_params=pltpu.CompilerParams(collective_id=0),
    )(x, dy)


# device time: 19460 ns/iter; 1.0135x vs baseline; 1.0071x over previous
import jax
import jax.numpy as jnp
from jax import lax
from jax.experimental import pallas as pl
from jax.experimental.pallas import tpu as pltpu

NC = 4


def kernel(x, dy):
    k, m = x.shape
    _, f = dy.shape
    m_half = m // 2
    f_half = f // 2
    fc = f_half // NC

    def body(x_ref, dy_ref, out_ref, c_send, rs_recv, p2_send, p2_recv,
             dy_half, x_keep, x_send, out_vmem,
             sems1_s, sems1_r, sems2_s, sems2_r, ldma_sems, xdma_sems,
             odma_sems):
        my_x = lax.axis_index("x")
        my_y = lax.axis_index("y")
        is_x0 = my_x == 0

        xs_copy = pltpu.make_async_copy(
            x_ref.at[:, pl.ds((1 - my_y) * m_half, m_half)], x_send,
            xdma_sems.at[0])
        xk_copy = pltpu.make_async_copy(
            x_ref.at[:, pl.ds(my_y * m_half, m_half)], x_keep,
            xdma_sems.at[1])
        xs_copy.start()
        xk_copy.start()
        dy_copies = []
        for c in range(NC):
            cp = pltpu.make_async_copy(
                dy_ref.at[:, pl.ds(my_x * f_half + c * fc, fc)],
                dy_half.at[:, c * fc:(c + 1) * fc], ldma_sems.at[c])
            cp.start()
            dy_copies.append(cp)

        barrier = pltpu.get_barrier_semaphore()
        pl.semaphore_signal(barrier, inc=1, device_id=(1 - my_x, my_y),
                            device_id_type=pl.DeviceIdType.MESH)
        pl.semaphore_signal(barrier, inc=1, device_id=(my_x, 1 - my_y),
                            device_id_type=pl.DeviceIdType.MESH)
        pl.semaphore_wait(barrier, 2)

        xs_copy.wait()
        rdma1 = []
        for c in range(NC):
            dy_copies[c].wait()
            c_send[c] = lax.dot_general(
                x_send[...], dy_half[:, c * fc:(c + 1) * fc],
                (((0,), (0,)), ((), ())),
                preferred_element_type=jnp.float32).astype(jnp.bfloat16)
            r = pltpu.make_async_remote_copy(
                src_ref=c_send.at[c], dst_ref=rs_recv.at[c],
                send_sem=sems1_s.at[c], recv_sem=sems1_r.at[c],
                device_id=(my_x, 1 - my_y),
                device_id_type=pl.DeviceIdType.MESH)
            r.start()
            rdma1.append(r)

        def drain(d):
            rdma2[d].wait_recv()
            lo, hi = d * fc, (d + 1) * fc

            @pl.when(is_x0)
            def _(lo=lo, hi=hi, d=d):
                out_vmem[:, f_half + lo:f_half + hi] = (
                    p2_recv[d].astype(jnp.float32))

            @pl.when(~is_x0)
            def _(lo=lo, hi=hi, d=d):
                out_vmem[:, lo:hi] = p2_recv[d].astype(jnp.float32)

            theirs = pl.ds((1 - my_x) * f_half + lo, fc)
            ocp = pltpu.make_async_copy(
                out_vmem.at[:, theirs], out_ref.at[:, theirs],
                odma_sems.at[NC + d])
            ocp.start()
            out_copies.append(ocp)
            rdma1[d].wait_send()
            rdma2[d].wait_send()

        LAG = 3
        xk_copy.wait()
        rdma2 = []
        out_copies = []
        for c in range(NC):
            keep = lax.dot_general(
                x_keep[...], dy_half[:, c * fc:(c + 1) * fc],
                (((0,), (0,)), ((), ())), preferred_element_type=jnp.float32)
            rdma1[c].wait_recv()
            val = keep + rs_recv[c].astype(jnp.float32)
            p2_send[c] = val.astype(jnp.bfloat16)
            lo, hi = c * fc, (c + 1) * fc

            @pl.when(is_x0)
            def _(lo=lo, hi=hi, val=val):
                out_vmem[:, lo:hi] = val

            @pl.when(~is_x0)
            def _(lo=lo, hi=hi, val=val):
                out_vmem[:, f_half + lo:f_half + hi] = val

            mine = pl.ds(my_x * f_half + lo, fc)
            ocp = pltpu.make_async_copy(
                out_vmem.at[:, mine], out_ref.at[:, mine], odma_sems.at[c])
            ocp.start()
            out_copies.append(ocp)

            r2 = pltpu.make_async_remote_copy(
                src_ref=p2_send.at[c], dst_ref=p2_recv.at[c],
                send_sem=sems2_s.at[c], recv_sem=sems2_r.at[c],
                device_id=(1 - my_x, my_y),
                device_id_type=pl.DeviceIdType.MESH)
            r2.start()
            rdma2.append(r2)
            if c >= LAG:
                drain(c - LAG)

        for d in range(NC - LAG, NC):
            drain(d)

        for ocp in out_copies:
            ocp.wait()

    return pl.pallas_call(
        body,
        out_shape=jax.ShapeDtypeStruct((m_half, f), jnp.float32),
        in_specs=[pl.BlockSpec(memory_space=pltpu.MemorySpace.HBM),
                  pl.BlockSpec(memory_space=pltpu.MemorySpace.HBM)],
        out_specs=pl.BlockSpec(memory_space=pltpu.MemorySpace.HBM),
        scratch_shapes=[
            pltpu.VMEM((NC, m_half, fc), jnp.bfloat16),
            pltpu.VMEM((NC, m_half, fc), jnp.bfloat16),
            pltpu.VMEM((NC, m_half, fc), jnp.bfloat16),
            pltpu.VMEM((NC, m_half, fc), jnp.bfloat16),
            pltpu.VMEM((k, f_half), jnp.float32),
            pltpu.VMEM((k, m_half), jnp.float32),
            pltpu.VMEM((k, m_half), jnp.float32),
            pltpu.VMEM((m_half, f), jnp.float32),
            pltpu.SemaphoreType.DMA((NC,)),
            pltpu.SemaphoreType.DMA((NC,)),
            pltpu.SemaphoreType.DMA((NC,)),
            pltpu.SemaphoreType.DMA((NC,)),
            pltpu.SemaphoreType.DMA((NC,)),
            pltpu.SemaphoreType.DMA((2,)),
            pltpu.SemaphoreType.DMA((2 * NC,)),
        ],
        compiler_params=pltpu.CompilerParams(collective_id=0),
    )(x, dy)
